# baseline (speedup 1.0000x reference)
import functools

import jax
import jax.numpy as jnp
from jax import lax
from jax.experimental import pallas as pl
from jax.experimental.pallas import tpu as pltpu
from jax.experimental.pallas import tpu_sc as plsc

N = 10000
E = 320000
D_IN = 128
D_HID = 64
LAMBDA = 1.0

L = 16
NC = 2
NS = 16
NW = NC * NS
EW = E // NW
NPAD = 10240
ZSL = NPAD // NS

_MESH = plsc.VectorSubcoreMesh(core_axis_name="c", subcore_axis_name="s")


def _enc_body(x_ref, wt_ref, v01_ref, h_ref, a_ref):
    h = jnp.dot(x_ref[...], wt_ref[...], preferred_element_type=jnp.float32)
    h_ref[...] = h
    a_ref[...] = jnp.dot(h, v01_ref[...], preferred_element_type=jnp.float32)


def _encode(x, W, v0, v1):
    v01 = jnp.concatenate([v0, v1], axis=1)
    h, a = pl.pallas_call(
        _enc_body,
        out_shape=(
            jax.ShapeDtypeStruct((N, D_HID), jnp.float32),
            jax.ShapeDtypeStruct((N, 2), jnp.float32),
        ),
    )(x, W.T, v01)
    return h, a[:, 0], a[:, 1]


@functools.partial(
    pl.kernel,
    out_type=(
        jax.ShapeDtypeStruct((E,), jnp.float32),
        jax.ShapeDtypeStruct((NC, NPAD), jnp.float32),
    ),
    mesh=_MESH,
    compiler_params=pltpu.CompilerParams(needs_layout_passes=False),
    scratch_types=[
        pltpu.VMEM((EW,), jnp.int32),
        pltpu.VMEM((EW,), jnp.int32),
        pltpu.VMEM((N,), jnp.float32),
        pltpu.VMEM((N,), jnp.float32),
        pltpu.VMEM((EW,), jnp.float32),
        pltpu.VMEM_SHARED((NPAD,), jnp.float32),
    ],
)
def _edge_scores(row_hbm, col_hbm, a0_hbm, a1_hbm, p_hbm, z_hbm,
                 row_v, col_v, a0_v, a1_v, p_v, z_sh):
    cid = lax.axis_index("c")
    sid = lax.axis_index("s")
    wid = cid * NS + sid
    base = wid * EW

    pltpu.sync_copy(row_hbm.at[pl.ds(base, EW)], row_v)
    pltpu.sync_copy(col_hbm.at[pl.ds(base, EW)], col_v)
    pltpu.sync_copy(a0_hbm, a0_v)
    pltpu.sync_copy(a1_hbm, a1_v)

    def zbody(i, _):
        p_v[pl.ds(i * L, L)] = jnp.zeros((L,), jnp.float32)
        return 0
    lax.fori_loop(0, ZSL // L, zbody, 0)
    pltpu.sync_copy(p_v.at[pl.ds(0, ZSL)], z_sh.at[pl.ds(sid * ZSL, ZSL)])
    plsc.subcore_barrier()

    def body(i, _):
        ir = row_v[pl.ds(i * L, L)]
        ic = col_v[pl.ds(i * L, L)]
        a0g = plsc.load_gather(a0_v, [ir])
        a1g = plsc.load_gather(a1_v, [ic])
        s = a0g + a1g
        sig = 1.0 / (1.0 + jnp.exp(-s))
        p_v[pl.ds(i * L, L)] = jnp.exp(sig)
        return 0
    lax.fori_loop(0, EW // L, body, 0)

    pltpu.sync_copy(p_v, p_hbm.at[pl.ds(base, EW)])
    pltpu.sync_copy(p_v, z_sh.at[row_v], add=True)
    plsc.subcore_barrier()
    pltpu.sync_copy(z_sh.at[pl.ds(sid * ZSL, ZSL)],
                    z_hbm.at[cid, pl.ds(sid * ZSL, ZSL)])


CH = 80
NCHUNK = EW // CH
NSL = NPAD // NS


def _make_msg_pass(compute_attn):
    out_acc = jax.ShapeDtypeStruct((NC, NPAD, D_HID), jnp.float32)
    out_type = ((jax.ShapeDtypeStruct((NW, NCHUNK, CH), jnp.float32), out_acc)
                if compute_attn else out_acc)
    scratch = [
        pltpu.VMEM((NCHUNK, CH), jnp.int32),
        pltpu.VMEM((NCHUNK, CH), jnp.int32),
        pltpu.VMEM((NCHUNK, CH), jnp.float32),
        pltpu.VMEM((CH, D_HID), jnp.float32),
        pltpu.VMEM((CH, D_HID), jnp.float32),
        pltpu.VMEM_SHARED((NPAD, D_HID), jnp.float32),
        pltpu.SemaphoreType.DMA,
        pltpu.SemaphoreType.DMA,
        pltpu.SemaphoreType.DMA,
        pltpu.SemaphoreType.DMA,
    ]
    if compute_attn:
        scratch += [
            pltpu.VMEM((NPAD,), jnp.float32),
            pltpu.VMEM((ZSL,), jnp.float32),
        ]

    def body(*refs):
        if compute_attn:
            (row_hbm, col_hbm, p_hbm, zp_hbm, tab_hbm, attn_hbm, acc_hbm,
             row2_v, col2_v, attn2_v, rows0_v, rows1_v, acc_sh,
             sg0, sg1, ss0, ss1, z_v, z2_v) = refs
        else:
            (row_hbm, col_hbm, attn_hbm, tab_hbm, acc_hbm,
             row2_v, col2_v, attn2_v, rows0_v, rows1_v, acc_sh,
             sg0, sg1, ss0, ss1) = refs
        rows_b = (rows0_v, rows1_v)
        sg = (sg0, sg1)
        ss = (ss0, ss1)
        cid = lax.axis_index("c")
        sid = lax.axis_index("s")
        wid = cid * NS + sid

        pltpu.sync_copy(row_hbm.at[wid], row2_v)
        pltpu.sync_copy(col_hbm.at[wid], col2_v)
        if compute_attn:
            pltpu.sync_copy(p_hbm.at[wid], attn2_v)
            pltpu.sync_copy(zp_hbm.at[0], z_v)

        def zrow(i, _):
            for d in range(D_HID // L):
                rows0_v[i, pl.ds(d * L, L)] = jnp.zeros((L,), jnp.float32)
            return 0
        lax.fori_loop(0, CH, zrow, 0)
        for j in range(NSL // CH):
            pltpu.sync_copy(rows0_v, acc_sh.at[pl.ds(sid * NSL + j * CH, CH)])

        if compute_attn:
            for j in range(NPAD // ZSL):
                pltpu.sync_copy(zp_hbm.at[1, pl.ds(j * ZSL, ZSL)], z2_v)
                def zsum(i, _):
                    sl = pl.ds(j * ZSL + i * L, L)
                    z_v[sl] = z_v[sl] + z2_v[pl.ds(i * L, L)]
                    return 0
                lax.fori_loop(0, ZSL // L, zsum, 0)
        plsc.subcore_barrier()

        pltpu.async_copy(tab_hbm.at[col2_v.at[0]], rows0_v, sg[0])

        if compute_attn:
            def avreg(k, _):
                def inner(i, _):
                    sl = pl.ds(i * L, L)
                    zg = plsc.load_gather(z_v, [row2_v[k, sl]])
                    attn2_v[k, sl] = attn2_v[k, sl] / zg
                    return 0
                lax.fori_loop(0, CH // L, inner, 0)
                return 0
            lax.fori_loop(0, NCHUNK, avreg, 0)
            pltpu.sync_copy(attn2_v, attn_hbm.at[wid])
        else:
            pltpu.sync_copy(attn_hbm.at[wid], attn2_v)

        def chunk_iter(k, _):
            parity = lax.rem(k, 2)
            for b in (0, 1):
                nb = 1 - b

                @pl.when(parity == b)
                def _(b=b, nb=nb):
                    rv = rows_b[b]
                    rn = rows_b[nb]

                    pltpu.make_async_copy(tab_hbm.at[col2_v.at[k]],
                                          rv, sg[b]).wait()

                    def scale(g, _):
                        av16 = attn2_v[k, pl.ds(g * L, L)]
                        e0 = g * L
                        for u in range(L):
                            av = jnp.full((L,), av16[u], jnp.float32)
                            for d in range(D_HID // L):
                                sl = pl.ds(d * L, L)
                                rv[e0 + u, sl] = rv[e0 + u, sl] * av
                        return 0
                    lax.fori_loop(0, CH // L, scale, 0)

                    @pl.when(k >= 1)
                    def _():
                        pltpu.make_async_copy(
                            tab_hbm.at[col2_v.at[0]], rn, ss[nb]).wait()
                    pltpu.async_copy(rv, acc_sh.at[row2_v.at[k]], ss[b],
                                     add=True)

                    @pl.when(k + 1 < NCHUNK)
                    def _():
                        pltpu.async_copy(
                            tab_hbm.at[col2_v.at[k + 1]], rn, sg[nb])
            return 0
        lax.fori_loop(0, NCHUNK, chunk_iter, 0)
        lastb = (NCHUNK - 1) & 1
        pltpu.make_async_copy(tab_hbm.at[col2_v.at[0]],
                              rows_b[lastb], ss[lastb]).wait()

        plsc.subcore_barrier()
        pltpu.sync_copy(acc_sh.at[pl.ds(sid * NSL, NSL)],
                        acc_hbm.at[cid, pl.ds(sid * NSL, NSL)])

    return pl.kernel(
        body,
        out_type=out_type,
        mesh=_MESH,
        compiler_params=pltpu.CompilerParams(
            needs_layout_passes=False, use_tc_tiling_on_sc=False),
        scratch_types=scratch,
    )


_msg_pass_attn = _make_msg_pass(True)
_msg_pass = _make_msg_pass(False)


def _sum_body(a_ref, b_ref, o_ref):
    o_ref[...] = a_ref[:N] + b_ref[:N]


def _sum_parts(parts):
    d = parts.shape[2]
    return pl.pallas_call(
        _sum_body,
        out_shape=jax.ShapeDtypeStruct((N, d), jnp.float32),
    )(parts[0], parts[1])


@functools.partial(
    pl.kernel,
    out_type=jax.ShapeDtypeStruct((NW, NCHUNK, CH), jnp.float32),
    mesh=_MESH,
    compiler_params=pltpu.CompilerParams(
        needs_layout_passes=False, use_tc_tiling_on_sc=False),
    scratch_types=[
        pltpu.VMEM((NCHUNK, CH), jnp.int32),
        pltpu.VMEM((NCHUNK, CH), jnp.int32),
        pltpu.VMEM((CH, D_HID), jnp.float32),
        pltpu.VMEM((CH, D_HID), jnp.float32),
        pltpu.VMEM((CH, D_HID), jnp.float32),
        pltpu.VMEM((CH, D_HID), jnp.float32),
        pltpu.VMEM((CH, L), jnp.float32),
        pltpu.VMEM((NCHUNK, CH), jnp.float32),
        pltpu.SemaphoreType.DMA,
        pltpu.SemaphoreType.DMA,
        pltpu.SemaphoreType.DMA,
        pltpu.SemaphoreType.DMA,
    ],
)
def _pair_dots(sr_hbm, sc_hbm, tab_hbm, dots_hbm,
               ia2_v, ib2_v, rowsa0_v, rowsa1_v, rowsb0_v, rowsb1_v,
               part_v, dots2_v, sa0, sa1, sb0, sb1):
    rowsa_b = (rowsa0_v, rowsa1_v)
    rowsb_b = (rowsb0_v, rowsb1_v)
    sa = (sa0, sa1)
    sb = (sb0, sb1)
    cid = lax.axis_index("c")
    sid = lax.axis_index("s")
    wid = cid * NS + sid
    lanes = lax.iota(jnp.int32, L)

    pltpu.sync_copy(sr_hbm.at[wid], ia2_v)
    pltpu.sync_copy(sc_hbm.at[wid], ib2_v)
    pltpu.async_copy(tab_hbm.at[ia2_v.at[0]], rowsa0_v, sa[0])
    pltpu.async_copy(tab_hbm.at[ib2_v.at[0]], rowsb0_v, sb[0])

    def chunk_iter(k, _):
        parity = lax.rem(k, 2)
        for b in (0, 1):
            nb = 1 - b

            @pl.when(parity == b)
            def _(b=b, nb=nb):
                ra = rowsa_b[b]
                rb = rowsb_b[b]

                @pl.when(k + 1 < NCHUNK)
                def _():
                    pltpu.async_copy(tab_hbm.at[ia2_v.at[k + 1]],
                                     rowsa_b[nb], sa[nb])
                    pltpu.async_copy(tab_hbm.at[ib2_v.at[k + 1]],
                                     rowsb_b[nb], sb[nb])
                pltpu.make_async_copy(tab_hbm.at[ia2_v.at[0]], ra, sa[b]).wait()
                pltpu.make_async_copy(tab_hbm.at[ib2_v.at[0]], rb, sb[b]).wait()

                def ppair(q, _):
                    for u in range(4):
                        e = q * 4 + u
                        t0 = ra[e, pl.ds(0, L)] * rb[e, pl.ds(0, L)]
                        t1 = ra[e, pl.ds(L, L)] * rb[e, pl.ds(L, L)]
                        t2 = ra[e, pl.ds(2 * L, L)] * rb[e, pl.ds(2 * L, L)]
                        t3 = ra[e, pl.ds(3 * L, L)] * rb[e, pl.ds(3 * L, L)]
                        part_v[e, :] = (t0 + t1) + (t2 + t3)
                    return 0
                lax.fori_loop(0, CH // 4, ppair, 0)

                def group(g, _):
                    re = g * L + lanes
                    acc0 = jnp.zeros((L,), jnp.float32)
                    acc1 = jnp.zeros((L,), jnp.float32)
                    for j in range(0, L, 2):
                        acc0 = acc0 + plsc.load_gather(
                            part_v, [re, jnp.full((L,), j, jnp.int32)])
                        acc1 = acc1 + plsc.load_gather(
                            part_v, [re, jnp.full((L,), j + 1, jnp.int32)])
                    dots2_v[k, pl.ds(g * L, L)] = acc0 + acc1
                    return 0
                lax.fori_loop(0, CH // L, group, 0)
        return 0
    lax.fori_loop(0, NCHUNK, chunk_iter, 0)
    pltpu.sync_copy(dots2_v, dots_hbm.at[wid])


def _final_body(gp_ref, w_ref, x_ref, dots_ref, xr_ref, tot_ref):
    g = gp_ref[0, :N] + gp_ref[1, :N]
    xr = jnp.dot(g, w_ref[...], preferred_element_type=jnp.float32)
    xr_ref[...] = xr
    diff = x_ref[...] - xr
    fl = jnp.sqrt(jnp.sum(diff * diff))
    d = dots_ref[...]
    sl = -jnp.sum(jnp.log(1.0 / (1.0 + jnp.exp(-d)) + 1e-08))
    tot_ref[0, 0] = fl + LAMBDA * sl


def _finalize(g_part, W, x, dots):
    xr, tot = pl.pallas_call(
        _final_body,
        out_shape=(
            jax.ShapeDtypeStruct((N, D_IN), jnp.float32),
            jax.ShapeDtypeStruct((1, 1), jnp.float32),
        ),
        out_specs=(
            pl.BlockSpec(memory_space=pltpu.VMEM),
            pl.BlockSpec(memory_space=pltpu.SMEM),
        ),
    )(g_part, W, x, dots.reshape(E // D_IN, D_IN))
    return xr, tot[0, 0]


def kernel(x, edge_index, structure_pairs, W, v0, v1):
    row, col = edge_index[0], edge_index[1]
    h, alpha0, alpha1 = _encode(x, W, v0, v1)

    p, z_part = _edge_scores(row, col, alpha0, alpha1)

    row3 = row.reshape(NW, NCHUNK, CH)
    col3 = col.reshape(NW, NCHUNK, CH)
    attn, henc_part = _msg_pass_attn(row3, col3, p.reshape(NW, NCHUNK, CH),
                                     z_part, h)
    h_enc = _sum_parts(henc_part)
    g_part = _msg_pass(row3, col3, attn, h_enc)
    sr, sc = structure_pairs[0], structure_pairs[1]
    dots = _pair_dots(sr.reshape(NW, NCHUNK, CH), sc.reshape(NW, NCHUNK, CH),
                      h_enc)
    x_recon, total_loss = _finalize(g_part, W, x, dots.reshape(E))
    return (total_loss, h_enc, x_recon)


# speedup vs baseline: 1.3024x; 1.3024x over previous
import functools

import jax
import jax.numpy as jnp
from jax import lax
from jax.experimental import pallas as pl
from jax.experimental.pallas import tpu as pltpu
from jax.experimental.pallas import tpu_sc as plsc

N = 10000
E = 320000
D_IN = 128
D_HID = 64
LAMBDA = 1.0

L = 16
NC = 2
NS = 16
NW = NC * NS
EW = E // NW
NPAD = 10240
ZSL = NPAD // NS

_MESH = plsc.VectorSubcoreMesh(core_axis_name="c", subcore_axis_name="s")


def _enc_body(x_ref, wt_ref, v01_ref, h_ref, a_ref):
    h = jnp.dot(x_ref[...], wt_ref[...], preferred_element_type=jnp.float32)
    h_ref[...] = h
    a_ref[...] = jnp.dot(h, v01_ref[...], preferred_element_type=jnp.float32)


def _encode(x, W, v0, v1):
    v01 = jnp.concatenate([v0, v1], axis=1)
    h, a = pl.pallas_call(
        _enc_body,
        out_shape=(
            jax.ShapeDtypeStruct((N, D_HID), jnp.float32),
            jax.ShapeDtypeStruct((N, 2), jnp.float32),
        ),
    )(x, W.T, v01)
    return h, a[:, 0], a[:, 1]


@functools.partial(
    pl.kernel,
    out_type=(
        jax.ShapeDtypeStruct((E,), jnp.float32),
        jax.ShapeDtypeStruct((NC, NPAD), jnp.float32),
    ),
    mesh=_MESH,
    compiler_params=pltpu.CompilerParams(needs_layout_passes=False),
    scratch_types=[
        pltpu.VMEM((EW,), jnp.int32),
        pltpu.VMEM((EW,), jnp.int32),
        pltpu.VMEM((N,), jnp.float32),
        pltpu.VMEM((N,), jnp.float32),
        pltpu.VMEM((EW,), jnp.float32),
        pltpu.VMEM_SHARED((NPAD,), jnp.float32),
    ],
)
def _edge_scores(row_hbm, col_hbm, a0_hbm, a1_hbm, p_hbm, z_hbm,
                 row_v, col_v, a0_v, a1_v, p_v, z_sh):
    cid = lax.axis_index("c")
    sid = lax.axis_index("s")
    wid = cid * NS + sid
    base = wid * EW

    pltpu.sync_copy(row_hbm.at[pl.ds(base, EW)], row_v)
    pltpu.sync_copy(col_hbm.at[pl.ds(base, EW)], col_v)
    pltpu.sync_copy(a0_hbm, a0_v)
    pltpu.sync_copy(a1_hbm, a1_v)

    def zbody(i, _):
        p_v[pl.ds(i * L, L)] = jnp.zeros((L,), jnp.float32)
        return 0
    lax.fori_loop(0, ZSL // L, zbody, 0)
    pltpu.sync_copy(p_v.at[pl.ds(0, ZSL)], z_sh.at[pl.ds(sid * ZSL, ZSL)])
    plsc.subcore_barrier()

    def body(i, _):
        ir = row_v[pl.ds(i * L, L)]
        ic = col_v[pl.ds(i * L, L)]
        a0g = plsc.load_gather(a0_v, [ir])
        a1g = plsc.load_gather(a1_v, [ic])
        s = a0g + a1g
        sig = 1.0 / (1.0 + jnp.exp(-s))
        p_v[pl.ds(i * L, L)] = jnp.exp(sig)
        return 0
    lax.fori_loop(0, EW // L, body, 0)

    pltpu.sync_copy(p_v, p_hbm.at[pl.ds(base, EW)])
    pltpu.sync_copy(p_v, z_sh.at[row_v], add=True)
    plsc.subcore_barrier()
    pltpu.sync_copy(z_sh.at[pl.ds(sid * ZSL, ZSL)],
                    z_hbm.at[cid, pl.ds(sid * ZSL, ZSL)])


CH = 80
NCHUNK = EW // CH
NSL = NPAD // NS


def _make_msg_pass(compute_attn):
    out_acc = jax.ShapeDtypeStruct((NC, NPAD, D_HID), jnp.float32)
    out_type = ((jax.ShapeDtypeStruct((NW, NCHUNK, CH), jnp.float32), out_acc)
                if compute_attn else out_acc)
    scratch = [
        pltpu.VMEM((NCHUNK, CH), jnp.int32),
        pltpu.VMEM((NCHUNK, CH), jnp.int32),
        pltpu.VMEM((NCHUNK, CH), jnp.float32),
        pltpu.VMEM((CH, D_HID), jnp.float32),
        pltpu.VMEM((CH, D_HID), jnp.float32),
        pltpu.VMEM((CH, D_HID), jnp.float32),
        pltpu.VMEM((CH, D_HID), jnp.float32),
        pltpu.VMEM_SHARED((NPAD, D_HID), jnp.float32),
    ] + [pltpu.SemaphoreType.DMA] * 8
    if compute_attn:
        scratch += [
            pltpu.VMEM((NPAD,), jnp.float32),
            pltpu.VMEM((ZSL,), jnp.float32),
        ]

    def body(*refs):
        if compute_attn:
            (row_hbm, col_hbm, p_hbm, zp_hbm, tab_hbm, attn_hbm, acc_hbm,
             row2_v, col2_v, attn2_v, r0, r1, r2, r3, acc_sh,
             sg0, sg1, sg2, sg3, ss0, ss1, ss2, ss3, z_v, z2_v) = refs
        else:
            (row_hbm, col_hbm, attn_hbm, tab_hbm, acc_hbm,
             row2_v, col2_v, attn2_v, r0, r1, r2, r3, acc_sh,
             sg0, sg1, sg2, sg3, ss0, ss1, ss2, ss3) = refs
        rows_b = (r0, r1, r2, r3)
        rows0_v = r0
        sg = (sg0, sg1, sg2, sg3)
        ss = (ss0, ss1, ss2, ss3)
        cid = lax.axis_index("c")
        sid = lax.axis_index("s")
        wid = cid * NS + sid

        pltpu.sync_copy(row_hbm.at[wid], row2_v)
        pltpu.sync_copy(col_hbm.at[wid], col2_v)
        if compute_attn:
            pltpu.sync_copy(p_hbm.at[wid], attn2_v)
            pltpu.sync_copy(zp_hbm.at[0], z_v)

        def zrow(i, _):
            for d in range(D_HID // L):
                rows0_v[i, pl.ds(d * L, L)] = jnp.zeros((L,), jnp.float32)
            return 0
        lax.fori_loop(0, CH, zrow, 0)
        for j in range(NSL // CH):
            pltpu.sync_copy(rows0_v, acc_sh.at[pl.ds(sid * NSL + j * CH, CH)])

        if compute_attn:
            for j in range(NPAD // ZSL):
                pltpu.sync_copy(zp_hbm.at[1, pl.ds(j * ZSL, ZSL)], z2_v)
                def zsum(i, _):
                    sl = pl.ds(j * ZSL + i * L, L)
                    z_v[sl] = z_v[sl] + z2_v[pl.ds(i * L, L)]
                    return 0
                lax.fori_loop(0, ZSL // L, zsum, 0)
        plsc.subcore_barrier()

        pltpu.async_copy(tab_hbm.at[col2_v.at[0]], rows0_v, sg[0])

        if compute_attn:
            def avreg(k, _):
                def inner(i, _):
                    sl = pl.ds(i * L, L)
                    zg = plsc.load_gather(z_v, [row2_v[k, sl]])
                    attn2_v[k, sl] = attn2_v[k, sl] / zg
                    return 0
                lax.fori_loop(0, CH // L, inner, 0)
                return 0
            lax.fori_loop(0, NCHUNK, avreg, 0)
            pltpu.sync_copy(attn2_v, attn_hbm.at[wid])
        else:
            pltpu.sync_copy(attn_hbm.at[wid], attn2_v)

        def chunk_iter(k, _):
            parity = lax.rem(k, 4)
            for b in range(4):
                nb = (b + 1) % 4

                @pl.when(parity == b)
                def _(b=b, nb=nb):
                    rv = rows_b[b]
                    rn = rows_b[nb]

                    @pl.when(k + 1 < NCHUNK)
                    def _():
                        @pl.when(k >= 3)
                        def _():
                            pltpu.make_async_copy(
                                tab_hbm.at[col2_v.at[0]], rn, ss[nb]).wait()
                        pltpu.async_copy(
                            tab_hbm.at[col2_v.at[k + 1]], rn, sg[nb])

                    pltpu.make_async_copy(tab_hbm.at[col2_v.at[k]],
                                          rv, sg[b]).wait()

                    def scale(g, _):
                        av16 = attn2_v[k, pl.ds(g * L, L)]
                        e0 = g * L
                        for u in range(L):
                            av = jnp.full((L,), av16[u], jnp.float32)
                            for d in range(D_HID // L):
                                sl = pl.ds(d * L, L)
                                rv[e0 + u, sl] = rv[e0 + u, sl] * av
                        return 0
                    lax.fori_loop(0, CH // L, scale, 0)
                    pltpu.async_copy(rv, acc_sh.at[row2_v.at[k]], ss[b],
                                     add=True)
            return 0
        lax.fori_loop(0, NCHUNK, chunk_iter, 0)
        for j in range(NCHUNK - 4, NCHUNK):
            pltpu.make_async_copy(tab_hbm.at[col2_v.at[0]],
                                  rows_b[j % 4], ss[j % 4]).wait()

        plsc.subcore_barrier()
        pltpu.sync_copy(acc_sh.at[pl.ds(sid * NSL, NSL)],
                        acc_hbm.at[cid, pl.ds(sid * NSL, NSL)])

    return pl.kernel(
        body,
        out_type=out_type,
        mesh=_MESH,
        compiler_params=pltpu.CompilerParams(
            needs_layout_passes=False, use_tc_tiling_on_sc=False),
        scratch_types=scratch,
    )


_msg_pass_attn = _make_msg_pass(True)
_msg_pass = _make_msg_pass(False)


def _sum_body(a_ref, b_ref, o_ref):
    o_ref[...] = a_ref[:N] + b_ref[:N]


def _sum_parts(parts):
    d = parts.shape[2]
    return pl.pallas_call(
        _sum_body,
        out_shape=jax.ShapeDtypeStruct((N, d), jnp.float32),
    )(parts[0], parts[1])


@functools.partial(
    pl.kernel,
    out_type=jax.ShapeDtypeStruct((NW, NCHUNK, CH), jnp.float32),
    mesh=_MESH,
    compiler_params=pltpu.CompilerParams(
        needs_layout_passes=False, use_tc_tiling_on_sc=False),
    scratch_types=[
        pltpu.VMEM((NCHUNK, CH), jnp.int32),
        pltpu.VMEM((NCHUNK, CH), jnp.int32),
        pltpu.VMEM((CH, D_HID), jnp.float32),
        pltpu.VMEM((CH, D_HID), jnp.float32),
        pltpu.VMEM((CH, D_HID), jnp.float32),
        pltpu.VMEM((CH, D_HID), jnp.float32),
        pltpu.VMEM((CH, L), jnp.float32),
        pltpu.VMEM((NCHUNK, CH), jnp.float32),
        pltpu.SemaphoreType.DMA,
        pltpu.SemaphoreType.DMA,
        pltpu.SemaphoreType.DMA,
        pltpu.SemaphoreType.DMA,
    ],
)
def _pair_dots(sr_hbm, sc_hbm, tab_hbm, dots_hbm,
               ia2_v, ib2_v, rowsa0_v, rowsa1_v, rowsb0_v, rowsb1_v,
               part_v, dots2_v, sa0, sa1, sb0, sb1):
    rowsa_b = (rowsa0_v, rowsa1_v)
    rowsb_b = (rowsb0_v, rowsb1_v)
    sa = (sa0, sa1)
    sb = (sb0, sb1)
    cid = lax.axis_index("c")
    sid = lax.axis_index("s")
    wid = cid * NS + sid
    lanes = lax.iota(jnp.int32, L)

    pltpu.sync_copy(sr_hbm.at[wid], ia2_v)
    pltpu.sync_copy(sc_hbm.at[wid], ib2_v)
    pltpu.async_copy(tab_hbm.at[ia2_v.at[0]], rowsa0_v, sa[0])
    pltpu.async_copy(tab_hbm.at[ib2_v.at[0]], rowsb0_v, sb[0])

    def chunk_iter(k, _):
        parity = lax.rem(k, 2)
        for b in (0, 1):
            nb = 1 - b

            @pl.when(parity == b)
            def _(b=b, nb=nb):
                ra = rowsa_b[b]
                rb = rowsb_b[b]

                @pl.when(k + 1 < NCHUNK)
                def _():
                    pltpu.async_copy(tab_hbm.at[ia2_v.at[k + 1]],
                                     rowsa_b[nb], sa[nb])
                    pltpu.async_copy(tab_hbm.at[ib2_v.at[k + 1]],
                                     rowsb_b[nb], sb[nb])
                pltpu.make_async_copy(tab_hbm.at[ia2_v.at[0]], ra, sa[b]).wait()
                pltpu.make_async_copy(tab_hbm.at[ib2_v.at[0]], rb, sb[b]).wait()

                def ppair(q, _):
                    for u in range(4):
                        e = q * 4 + u
                        t0 = ra[e, pl.ds(0, L)] * rb[e, pl.ds(0, L)]
                        t1 = ra[e, pl.ds(L, L)] * rb[e, pl.ds(L, L)]
                        t2 = ra[e, pl.ds(2 * L, L)] * rb[e, pl.ds(2 * L, L)]
                        t3 = ra[e, pl.ds(3 * L, L)] * rb[e, pl.ds(3 * L, L)]
                        part_v[e, :] = (t0 + t1) + (t2 + t3)
                    return 0
                lax.fori_loop(0, CH // 4, ppair, 0)

                def group(g, _):
                    re = g * L + lanes
                    acc0 = jnp.zeros((L,), jnp.float32)
                    acc1 = jnp.zeros((L,), jnp.float32)
                    for j in range(0, L, 2):
                        acc0 = acc0 + plsc.load_gather(
                            part_v, [re, jnp.full((L,), j, jnp.int32)])
                        acc1 = acc1 + plsc.load_gather(
                            part_v, [re, jnp.full((L,), j + 1, jnp.int32)])
                    dots2_v[k, pl.ds(g * L, L)] = acc0 + acc1
                    return 0
                lax.fori_loop(0, CH // L, group, 0)
        return 0
    lax.fori_loop(0, NCHUNK, chunk_iter, 0)
    pltpu.sync_copy(dots2_v, dots_hbm.at[wid])


def _final_body(gp_ref, w_ref, x_ref, dots_ref, xr_ref, tot_ref):
    g = gp_ref[0, :N] + gp_ref[1, :N]
    xr = jnp.dot(g, w_ref[...], preferred_element_type=jnp.float32)
    xr_ref[...] = xr
    diff = x_ref[...] - xr
    fl = jnp.sqrt(jnp.sum(diff * diff))
    d = dots_ref[...]
    sl = -jnp.sum(jnp.log(1.0 / (1.0 + jnp.exp(-d)) + 1e-08))
    tot_ref[0, 0] = fl + LAMBDA * sl


def _finalize(g_part, W, x, dots):
    xr, tot = pl.pallas_call(
        _final_body,
        out_shape=(
            jax.ShapeDtypeStruct((N, D_IN), jnp.float32),
            jax.ShapeDtypeStruct((1, 1), jnp.float32),
        ),
        out_specs=(
            pl.BlockSpec(memory_space=pltpu.VMEM),
            pl.BlockSpec(memory_space=pltpu.SMEM),
        ),
    )(g_part, W, x, dots.reshape(E // D_IN, D_IN))
    return xr, tot[0, 0]


def kernel(x, edge_index, structure_pairs, W, v0, v1):
    row, col = edge_index[0], edge_index[1]
    h, alpha0, alpha1 = _encode(x, W, v0, v1)

    p, z_part = _edge_scores(row, col, alpha0, alpha1)

    row3 = row.reshape(NW, NCHUNK, CH)
    col3 = col.reshape(NW, NCHUNK, CH)
    attn, henc_part = _msg_pass_attn(row3, col3, p.reshape(NW, NCHUNK, CH),
                                     z_part, h)
    h_enc = _sum_parts(henc_part)
    g_part = _msg_pass(row3, col3, attn, h_enc)
    sr, sc = structure_pairs[0], structure_pairs[1]
    dots = _pair_dots(sr.reshape(NW, NCHUNK, CH), sc.reshape(NW, NCHUNK, CH),
                      h_enc)
    x_recon, total_loss = _finalize(g_part, W, x, dots.reshape(E))
    return (total_loss, h_enc, x_recon)


# speedup vs baseline: 1.3384x; 1.0276x over previous
import functools

import jax
import jax.numpy as jnp
from jax import lax
from jax.experimental import pallas as pl
from jax.experimental.pallas import tpu as pltpu
from jax.experimental.pallas import tpu_sc as plsc

N = 10000
E = 320000
D_IN = 128
D_HID = 64
LAMBDA = 1.0

L = 16
NC = 2
NS = 16
NW = NC * NS
EW = E // NW
NPAD = 10240
ZSL = NPAD // NS

_MESH = plsc.VectorSubcoreMesh(core_axis_name="c", subcore_axis_name="s")


def _enc_body(x_ref, wt_ref, v01_ref, h_ref, a_ref):
    h = jnp.dot(x_ref[...], wt_ref[...], preferred_element_type=jnp.float32)
    h_ref[...] = h
    a_ref[...] = jnp.dot(h, v01_ref[...], preferred_element_type=jnp.float32)


def _encode(x, W, v0, v1):
    v01 = jnp.concatenate([v0, v1], axis=1)
    h, a = pl.pallas_call(
        _enc_body,
        out_shape=(
            jax.ShapeDtypeStruct((N, D_HID), jnp.float32),
            jax.ShapeDtypeStruct((N, 2), jnp.float32),
        ),
    )(x, W.T, v01)
    return h, a[:, 0], a[:, 1]


@functools.partial(
    pl.kernel,
    out_type=(
        jax.ShapeDtypeStruct((E,), jnp.float32),
        jax.ShapeDtypeStruct((NC, NPAD), jnp.float32),
    ),
    mesh=_MESH,
    compiler_params=pltpu.CompilerParams(needs_layout_passes=False),
    scratch_types=[
        pltpu.VMEM((EW,), jnp.int32),
        pltpu.VMEM((EW,), jnp.int32),
        pltpu.VMEM((N,), jnp.float32),
        pltpu.VMEM((N,), jnp.float32),
        pltpu.VMEM((EW,), jnp.float32),
        pltpu.VMEM_SHARED((NPAD,), jnp.float32),
    ],
)
def _edge_scores(row_hbm, col_hbm, a0_hbm, a1_hbm, p_hbm, z_hbm,
                 row_v, col_v, a0_v, a1_v, p_v, z_sh):
    cid = lax.axis_index("c")
    sid = lax.axis_index("s")
    wid = cid * NS + sid
    base = wid * EW

    pltpu.sync_copy(row_hbm.at[pl.ds(base, EW)], row_v)
    pltpu.sync_copy(col_hbm.at[pl.ds(base, EW)], col_v)
    pltpu.sync_copy(a0_hbm, a0_v)
    pltpu.sync_copy(a1_hbm, a1_v)

    def zbody(i, _):
        p_v[pl.ds(i * L, L)] = jnp.zeros((L,), jnp.float32)
        return 0
    lax.fori_loop(0, ZSL // L, zbody, 0)
    pltpu.sync_copy(p_v.at[pl.ds(0, ZSL)], z_sh.at[pl.ds(sid * ZSL, ZSL)])
    plsc.subcore_barrier()

    def body(i, _):
        ir = row_v[pl.ds(i * L, L)]
        ic = col_v[pl.ds(i * L, L)]
        a0g = plsc.load_gather(a0_v, [ir])
        a1g = plsc.load_gather(a1_v, [ic])
        s = a0g + a1g
        sig = 1.0 / (1.0 + jnp.exp(-s))
        p_v[pl.ds(i * L, L)] = jnp.exp(sig)
        return 0
    lax.fori_loop(0, EW // L, body, 0)

    pltpu.sync_copy(p_v, p_hbm.at[pl.ds(base, EW)])
    pltpu.sync_copy(p_v, z_sh.at[row_v], add=True)
    plsc.subcore_barrier()
    pltpu.sync_copy(z_sh.at[pl.ds(sid * ZSL, ZSL)],
                    z_hbm.at[cid, pl.ds(sid * ZSL, ZSL)])


CH = 80
NCHUNK = EW // CH
NSL = NPAD // NS


def _make_msg_pass(compute_attn):
    out_acc = jax.ShapeDtypeStruct((NC, NPAD, D_HID), jnp.float32)
    out_type = ((jax.ShapeDtypeStruct((NW, NCHUNK, CH), jnp.float32), out_acc)
                if compute_attn else out_acc)
    scratch = [
        pltpu.VMEM((NCHUNK, CH), jnp.int32),
        pltpu.VMEM((NCHUNK, CH), jnp.int32),
        pltpu.VMEM((NCHUNK, CH), jnp.float32),
        pltpu.VMEM((CH, D_HID), jnp.float32),
        pltpu.VMEM((CH, D_HID), jnp.float32),
        pltpu.VMEM((CH, D_HID), jnp.float32),
        pltpu.VMEM((CH, D_HID), jnp.float32),
        pltpu.VMEM_SHARED((NPAD, D_HID), jnp.float32),
    ] + [pltpu.SemaphoreType.DMA] * 8
    if compute_attn:
        scratch += [
            pltpu.VMEM((NPAD,), jnp.float32),
            pltpu.VMEM((ZSL,), jnp.float32),
        ]

    def body(*refs):
        if compute_attn:
            (row_hbm, col_hbm, p_hbm, zp_hbm, tab_hbm, attn_hbm, acc_hbm,
             row2_v, col2_v, attn2_v, r0, r1, r2, r3, acc_sh,
             sg0, sg1, sg2, sg3, ss0, ss1, ss2, ss3, z_v, z2_v) = refs
        else:
            (row_hbm, col_hbm, attn_hbm, tab_hbm, acc_hbm,
             row2_v, col2_v, attn2_v, r0, r1, r2, r3, acc_sh,
             sg0, sg1, sg2, sg3, ss0, ss1, ss2, ss3) = refs
        rows_b = (r0, r1, r2, r3)
        rows0_v = r0
        sg = (sg0, sg1, sg2, sg3)
        ss = (ss0, ss1, ss2, ss3)
        cid = lax.axis_index("c")
        sid = lax.axis_index("s")
        wid = cid * NS + sid

        pltpu.sync_copy(row_hbm.at[wid], row2_v)
        pltpu.sync_copy(col_hbm.at[wid], col2_v)
        if compute_attn:
            pltpu.sync_copy(p_hbm.at[wid], attn2_v)
            pltpu.sync_copy(zp_hbm.at[0], z_v)

        def zrow(i, _):
            for d in range(D_HID // L):
                rows0_v[i, pl.ds(d * L, L)] = jnp.zeros((L,), jnp.float32)
            return 0
        lax.fori_loop(0, CH, zrow, 0)
        for j in range(NSL // CH):
            pltpu.sync_copy(rows0_v, acc_sh.at[pl.ds(sid * NSL + j * CH, CH)])

        if compute_attn:
            for j in range(NPAD // ZSL):
                pltpu.sync_copy(zp_hbm.at[1, pl.ds(j * ZSL, ZSL)], z2_v)
                def zsum(i, _):
                    sl = pl.ds(j * ZSL + i * L, L)
                    z_v[sl] = z_v[sl] + z2_v[pl.ds(i * L, L)]
                    return 0
                lax.fori_loop(0, ZSL // L, zsum, 0)
        plsc.subcore_barrier()

        pltpu.async_copy(tab_hbm.at[col2_v.at[0]], rows0_v, sg[0])

        if compute_attn:
            def avreg(k, _):
                def inner(i, _):
                    sl = pl.ds(i * L, L)
                    zg = plsc.load_gather(z_v, [row2_v[k, sl]])
                    attn2_v[k, sl] = attn2_v[k, sl] / zg
                    return 0
                lax.fori_loop(0, CH // L, inner, 0)
                return 0
            lax.fori_loop(0, NCHUNK, avreg, 0)
            pltpu.sync_copy(attn2_v, attn_hbm.at[wid])
        else:
            pltpu.sync_copy(attn_hbm.at[wid], attn2_v)

        def chunk_iter(k, _):
            parity = lax.rem(k, 4)
            for b in range(4):
                nb = (b + 1) % 4

                @pl.when(parity == b)
                def _(b=b, nb=nb):
                    rv = rows_b[b]
                    rn = rows_b[nb]

                    @pl.when(k + 1 < NCHUNK)
                    def _():
                        @pl.when(k >= 3)
                        def _():
                            pltpu.make_async_copy(
                                tab_hbm.at[col2_v.at[0]], rn, ss[nb]).wait()
                        pltpu.async_copy(
                            tab_hbm.at[col2_v.at[k + 1]], rn, sg[nb])

                    pltpu.make_async_copy(tab_hbm.at[col2_v.at[k]],
                                          rv, sg[b]).wait()

                    def scale(g, _):
                        av16 = attn2_v[k, pl.ds(g * L, L)]
                        e0 = g * L
                        for u in range(L):
                            av = jnp.full((L,), av16[u], jnp.float32)
                            for d in range(D_HID // L):
                                sl = pl.ds(d * L, L)
                                rv[e0 + u, sl] = rv[e0 + u, sl] * av
                        return 0
                    lax.fori_loop(0, CH // L, scale, 0)
                    pltpu.async_copy(rv, acc_sh.at[row2_v.at[k]], ss[b],
                                     add=True)
            return 0
        lax.fori_loop(0, NCHUNK, chunk_iter, 0)
        for j in range(NCHUNK - 4, NCHUNK):
            pltpu.make_async_copy(tab_hbm.at[col2_v.at[0]],
                                  rows_b[j % 4], ss[j % 4]).wait()

        plsc.subcore_barrier()
        pltpu.sync_copy(acc_sh.at[pl.ds(sid * NSL, NSL)],
                        acc_hbm.at[cid, pl.ds(sid * NSL, NSL)])

    return pl.kernel(
        body,
        out_type=out_type,
        mesh=_MESH,
        compiler_params=pltpu.CompilerParams(
            needs_layout_passes=False, use_tc_tiling_on_sc=False),
        scratch_types=scratch,
    )


_msg_pass_attn = _make_msg_pass(True)
_msg_pass = _make_msg_pass(False)


def _sum_body(a_ref, b_ref, o_ref):
    o_ref[...] = a_ref[:N] + b_ref[:N]


def _sum_parts(parts):
    d = parts.shape[2]
    return pl.pallas_call(
        _sum_body,
        out_shape=jax.ShapeDtypeStruct((N, d), jnp.float32),
    )(parts[0], parts[1])


@functools.partial(
    pl.kernel,
    out_type=jax.ShapeDtypeStruct((NW, NCHUNK, CH), jnp.float32),
    mesh=_MESH,
    compiler_params=pltpu.CompilerParams(
        needs_layout_passes=False, use_tc_tiling_on_sc=False),
    scratch_types=[
        pltpu.VMEM((NCHUNK, CH), jnp.int32),
        pltpu.VMEM((NCHUNK, CH), jnp.int32),
        pltpu.VMEM((CH, D_HID), jnp.bfloat16),
        pltpu.VMEM((CH, D_HID), jnp.bfloat16),
        pltpu.VMEM((CH, D_HID), jnp.bfloat16),
        pltpu.VMEM((CH, D_HID), jnp.bfloat16),
        pltpu.VMEM((CH, L), jnp.float32),
        pltpu.VMEM((NCHUNK, CH), jnp.float32),
        pltpu.SemaphoreType.DMA,
        pltpu.SemaphoreType.DMA,
        pltpu.SemaphoreType.DMA,
        pltpu.SemaphoreType.DMA,
    ],
)
def _pair_dots(sr_hbm, sc_hbm, tab_hbm, dots_hbm,
               ia2_v, ib2_v, rowsa0_v, rowsa1_v, rowsb0_v, rowsb1_v,
               part_v, dots2_v, sa0, sa1, sb0, sb1):
    rowsa_b = (rowsa0_v, rowsa1_v)
    rowsb_b = (rowsb0_v, rowsb1_v)
    sa = (sa0, sa1)
    sb = (sb0, sb1)
    cid = lax.axis_index("c")
    sid = lax.axis_index("s")
    wid = cid * NS + sid
    lanes = lax.iota(jnp.int32, L)

    pltpu.sync_copy(sr_hbm.at[wid], ia2_v)
    pltpu.sync_copy(sc_hbm.at[wid], ib2_v)
    pltpu.async_copy(tab_hbm.at[ia2_v.at[0]], rowsa0_v, sa[0])
    pltpu.async_copy(tab_hbm.at[ib2_v.at[0]], rowsb0_v, sb[0])

    def chunk_iter(k, _):
        parity = lax.rem(k, 2)
        for b in (0, 1):
            nb = 1 - b

            @pl.when(parity == b)
            def _(b=b, nb=nb):
                ra = rowsa_b[b]
                rb = rowsb_b[b]

                @pl.when(k + 1 < NCHUNK)
                def _():
                    pltpu.async_copy(tab_hbm.at[ia2_v.at[k + 1]],
                                     rowsa_b[nb], sa[nb])
                    pltpu.async_copy(tab_hbm.at[ib2_v.at[k + 1]],
                                     rowsb_b[nb], sb[nb])
                pltpu.make_async_copy(tab_hbm.at[ia2_v.at[0]], ra, sa[b]).wait()
                pltpu.make_async_copy(tab_hbm.at[ib2_v.at[0]], rb, sb[b]).wait()

                def ppair(q, _):
                    for u in range(4):
                        e = q * 4 + u
                        pa0 = ra[e, pl.ds(0, 2 * L)]
                        pb0 = rb[e, pl.ds(0, 2 * L)]
                        pa1 = ra[e, pl.ds(2 * L, 2 * L)]
                        pb1 = rb[e, pl.ds(2 * L, 2 * L)]
                        a0, a1 = plsc.unpack(pa0, format=plsc.PackFormat.INTERLEAVED)
                        b0, b1 = plsc.unpack(pb0, format=plsc.PackFormat.INTERLEAVED)
                        a2, a3 = plsc.unpack(pa1, format=plsc.PackFormat.INTERLEAVED)
                        b2, b3 = plsc.unpack(pb1, format=plsc.PackFormat.INTERLEAVED)
                        t0 = a0 * b0
                        t1 = a1 * b1
                        t2 = a2 * b2
                        t3 = a3 * b3
                        part_v[e, :] = (t0 + t1) + (t2 + t3)
                    return 0
                lax.fori_loop(0, CH // 4, ppair, 0)

                def group(g, _):
                    re = g * L + lanes
                    acc0 = jnp.zeros((L,), jnp.float32)
                    acc1 = jnp.zeros((L,), jnp.float32)
                    for j in range(0, L, 2):
                        acc0 = acc0 + plsc.load_gather(
                            part_v, [re, jnp.full((L,), j, jnp.int32)])
                        acc1 = acc1 + plsc.load_gather(
                            part_v, [re, jnp.full((L,), j + 1, jnp.int32)])
                    dots2_v[k, pl.ds(g * L, L)] = acc0 + acc1
                    return 0
                lax.fori_loop(0, CH // L, group, 0)
        return 0
    lax.fori_loop(0, NCHUNK, chunk_iter, 0)
    pltpu.sync_copy(dots2_v, dots_hbm.at[wid])


def _final_body(gp_ref, w_ref, x_ref, dots_ref, xr_ref, tot_ref):
    g = gp_ref[0, :N] + gp_ref[1, :N]
    xr = jnp.dot(g, w_ref[...], preferred_element_type=jnp.float32)
    xr_ref[...] = xr
    diff = x_ref[...] - xr
    fl = jnp.sqrt(jnp.sum(diff * diff))
    d = dots_ref[...]
    sl = -jnp.sum(jnp.log(1.0 / (1.0 + jnp.exp(-d)) + 1e-08))
    tot_ref[0, 0] = fl + LAMBDA * sl


def _finalize(g_part, W, x, dots):
    xr, tot = pl.pallas_call(
        _final_body,
        out_shape=(
            jax.ShapeDtypeStruct((N, D_IN), jnp.float32),
            jax.ShapeDtypeStruct((1, 1), jnp.float32),
        ),
        out_specs=(
            pl.BlockSpec(memory_space=pltpu.VMEM),
            pl.BlockSpec(memory_space=pltpu.SMEM),
        ),
    )(g_part, W, x, dots.reshape(E // D_IN, D_IN))
    return xr, tot[0, 0]


def kernel(x, edge_index, structure_pairs, W, v0, v1):
    row, col = edge_index[0], edge_index[1]
    h, alpha0, alpha1 = _encode(x, W, v0, v1)

    p, z_part = _edge_scores(row, col, alpha0, alpha1)

    row3 = row.reshape(NW, NCHUNK, CH)
    col3 = col.reshape(NW, NCHUNK, CH)
    attn, henc_part = _msg_pass_attn(row3, col3, p.reshape(NW, NCHUNK, CH),
                                     z_part, h)
    h_enc = _sum_parts(henc_part)
    g_part = _msg_pass(row3, col3, attn, h_enc)
    sr, sc = structure_pairs[0], structure_pairs[1]
    dots = _pair_dots(sr.reshape(NW, NCHUNK, CH), sc.reshape(NW, NCHUNK, CH),
                      h_enc.astype(jnp.bfloat16))
    x_recon, total_loss = _finalize(g_part, W, x, dots.reshape(E))
    return (total_loss, h_enc, x_recon)
